# baseline (speedup 1.0000x reference)
import functools

import jax
import jax.numpy as jnp
from jax import lax
from jax.experimental import pallas as pl
from jax.experimental.pallas import tpu as pltpu
from jax.experimental.pallas import tpu_sc as plsc

_N, _D, _E = 10000, 128, 320000
_CHUNK = 128
_NBUF = 2
_PC = 40
_ROWS_PER_TILE = 640


def _sc_scatter_add(x, src3, dst3):
    info = plsc.get_sparse_core_info()
    nc, ns = info.num_cores, info.num_subcores
    nw = nc * ns
    assert src3.shape[0] == nw and src3.shape[2] == _CHUNK
    cpw = src3.shape[1]
    assert cpw % _PC == 0 and _PC % _NBUF == 0 and _PC % 8 == 0
    n_pad = _ROWS_PER_TILE * ns
    zp = _ROWS_PER_TILE // _CHUNK
    assert _ROWS_PER_TILE % _CHUNK == 0
    tail_rows = _N - (ns - 1) * _ROWS_PER_TILE
    assert tail_rows > 0 and (ns - 1) * _ROWS_PER_TILE % 8 == 0

    mesh = plsc.VectorSubcoreMesh(core_axis_name="c", subcore_axis_name="s")

    @functools.partial(
        pl.kernel,
        out_type=jax.ShapeDtypeStruct((nc, _N, _D), jnp.float32),
        mesh=mesh,
        scratch_types=[
            pltpu.MemorySpace.VMEM_SHARED((n_pad, _D), jnp.float32),
            pltpu.MemorySpace.VMEM((_PC, _CHUNK), jnp.int32),
            pltpu.MemorySpace.VMEM((_PC, _CHUNK), jnp.int32),
            [pltpu.MemorySpace.VMEM((_CHUNK, _D), jnp.float32)] * _NBUF,
            [pltpu.SemaphoreType.DMA] * _NBUF,
        ],
    )
    def k(x_hbm, src_hbm, dst_hbm, out_hbm, agg_sh, src_v, dst_v, rows, sems):
        c = lax.axis_index("c")
        s = lax.axis_index("s")
        wid = s * nc + c

        def zrow(i, _):
            def zlane(j, _):
                rows[0][i, pl.ds(j * 16, 16)] = jnp.zeros((16,), jnp.float32)
                return 0

            lax.fori_loop(0, _D // 16, zlane, 0)
            return 0

        lax.fori_loop(0, _CHUNK, zrow, 0)

        for p in range(zp):
            pltpu.sync_copy(
                rows[0],
                agg_sh.at[pl.ds(s * _ROWS_PER_TILE + p * _CHUNK, _CHUNK)],
            )

        plsc.subcore_barrier()

        def phase(p, _):
            pltpu.sync_copy(src_hbm.at[wid, pl.ds(p * _PC, _PC)], src_v)
            pltpu.sync_copy(dst_hbm.at[wid, pl.ds(p * _PC, _PC)], dst_v)

            def body(i, _):
                ds = [
                    pltpu.async_copy(x_hbm.at[src_v.at[i * _NBUF + b]],
                                     rows[b], sems[b])
                    for b in range(_NBUF)
                ]
                for b in range(_NBUF):
                    ds[b].wait()
                    pltpu.sync_copy(rows[b],
                                    agg_sh.at[dst_v.at[i * _NBUF + b]],
                                    add=True)
                return 0

            lax.fori_loop(0, _PC // _NBUF, body, 0)
            return 0

        lax.fori_loop(0, cpw // _PC, phase, 0)
        plsc.subcore_barrier()

        @pl.when(s < ns - 1)
        def _():
            pltpu.sync_copy(
                agg_sh.at[pl.ds(s * _ROWS_PER_TILE, _ROWS_PER_TILE)],
                out_hbm.at[c, pl.ds(s * _ROWS_PER_TILE, _ROWS_PER_TILE)],
            )

        @pl.when(s == ns - 1)
        def _():
            pltpu.sync_copy(
                agg_sh.at[pl.ds((ns - 1) * _ROWS_PER_TILE, tail_rows)],
                out_hbm.at[c, pl.ds((ns - 1) * _ROWS_PER_TILE, tail_rows)],
            )

    return k(x, src3, dst3)


def _mlp_body(x_ref, p_ref, w1_ref, b1_ref, w2_ref, b2_ref, wg_ref, bg_ref,
              wa_ref, ba_ref, h_ref, a_ref):
    dn = (((1,), (1,)), ((), ()))
    xa = x_ref[...] + p_ref[0] + p_ref[1]
    h1 = lax.dot_general(xa, w1_ref[...], dn, preferred_element_type=jnp.float32)
    h1 = jnp.maximum(h1 + b1_ref[...], 0.0)
    h = lax.dot_general(h1, w2_ref[...], dn, preferred_element_type=jnp.float32)
    h = h + b2_ref[...]
    h_ref[...] = h
    ga = lax.dot_general(h, wg_ref[...], dn, preferred_element_type=jnp.float32)
    ga = jnp.tanh(ga + bg_ref[...])
    alpha = lax.dot_general(wa_ref[...], ga, dn, preferred_element_type=jnp.float32)
    e = jnp.exp(alpha - jnp.max(alpha))
    a_ref[...] = e / jnp.sum(e)


def kernel(x, edge_index, W1, b1, W2, b2, Wg, bg, Wa, ba):
    src = edge_index[0]
    dst = edge_index[1]
    info = plsc.get_sparse_core_info()
    nw = info.num_cores * info.num_subcores
    n_pad = _ROWS_PER_TILE * info.num_subcores
    epw = _E // nw
    cpw = epw // 125
    assert epw % 125 == 0 and cpw % _PC == 0
    src3 = jnp.pad(src.reshape(nw, cpw, 125), ((0, 0), (0, 0), (0, 3)))
    pad_rows = (_N + jnp.arange(nw, dtype=dst.dtype))[:, None, None]
    dst3 = jnp.concatenate(
        [dst.reshape(nw, cpw, 125),
         jnp.broadcast_to(pad_rows, (nw, cpw, 3))], axis=2)
    parts = _sc_scatter_add(x, src3, dst3)
    h, a = pl.pallas_call(
        _mlp_body,
        out_shape=[
            jax.ShapeDtypeStruct((_N, _D), jnp.float32),
            jax.ShapeDtypeStruct((1, _N), jnp.float32),
        ],
    )(x, parts, W1, b1.reshape(1, -1), W2, b2.reshape(1, -1),
      Wg, bg.reshape(1, -1), Wa, ba.reshape(1, 1))
    return h, a[0]


# speedup vs baseline: 2.0500x; 2.0500x over previous
import functools

import jax
import jax.numpy as jnp
from jax import lax
from jax.experimental import pallas as pl
from jax.experimental.pallas import tpu as pltpu
from jax.experimental.pallas import tpu_sc as plsc

_N, _D, _E = 10000, 128, 320000
_CHUNK = 128
_NBUF = 2
_ROWS_PER_TILE = 640


def _sc_scatter_add(x, src, dst):
    info = plsc.get_sparse_core_info()
    nc, ns = info.num_cores, info.num_subcores
    nw = nc * ns
    n_chunks = _E // _CHUNK
    assert _E % _CHUNK == 0
    n_pad = _ROWS_PER_TILE * ns
    zp = _ROWS_PER_TILE // _CHUNK
    assert _ROWS_PER_TILE % _CHUNK == 0
    tail_rows = _N - (ns - 1) * _ROWS_PER_TILE
    assert tail_rows > 0 and (ns - 1) * _ROWS_PER_TILE % 8 == 0

    mesh = plsc.VectorSubcoreMesh(core_axis_name="c", subcore_axis_name="s")

    @functools.partial(
        pl.kernel,
        out_type=jax.ShapeDtypeStruct((nc, _N, _D), jnp.float32),
        mesh=mesh,
        scratch_types=[
            pltpu.MemorySpace.VMEM_SHARED((n_pad, _D), jnp.float32),
            [pltpu.MemorySpace.VMEM((_CHUNK,), jnp.int32)] * _NBUF,
            [pltpu.MemorySpace.VMEM((_CHUNK,), jnp.int32)] * _NBUF,
            [pltpu.MemorySpace.VMEM((_CHUNK, _D), jnp.float32)] * _NBUF,
            [pltpu.SemaphoreType.DMA] * _NBUF,
            [pltpu.SemaphoreType.DMA] * _NBUF,
            [pltpu.SemaphoreType.DMA] * _NBUF,
            [pltpu.SemaphoreType.DMA] * _NBUF,
        ],
    )
    def k(x_hbm, src_hbm, dst_hbm, out_hbm, agg_sh, src_v, dst_v, rows,
          sem_is, sem_id, sem_g, sem_s):
        c = lax.axis_index("c")
        s = lax.axis_index("s")
        wid = s * nc + c

        def zrow(i, _):
            def zlane(j, _):
                rows[0][i, pl.ds(j * 16, 16)] = jnp.zeros((16,), jnp.float32)
                return 0

            lax.fori_loop(0, _D // 16, zlane, 0)
            return 0

        lax.fori_loop(0, _CHUNK, zrow, 0)

        for p in range(zp):
            pltpu.sync_copy(
                rows[0],
                agg_sh.at[pl.ds(s * _ROWS_PER_TILE + p * _CHUNK, _CHUNK)],
            )

        plsc.subcore_barrier()

        base_count = n_chunks // nw
        extra = n_chunks % nw
        assert base_count % _NBUF == 0

        def body(i, _):
            idx_d = []
            for b in range(_NBUF):
                base = ((i * _NBUF + b) * nw + wid) * _CHUNK
                idx_d.append((
                    pltpu.async_copy(src_hbm.at[pl.ds(base, _CHUNK)],
                                     src_v[b], sem_is[b]),
                    pltpu.async_copy(dst_hbm.at[pl.ds(base, _CHUNK)],
                                     dst_v[b], sem_id[b]),
                ))
            g_d = []
            for b in range(_NBUF):
                idx_d[b][0].wait()
                g_d.append(pltpu.async_copy(x_hbm.at[src_v[b]], rows[b],
                                            sem_g[b]))
            s_d = []
            for b in range(_NBUF):
                g_d[b].wait()
                idx_d[b][1].wait()
                s_d.append(pltpu.async_copy(rows[b], agg_sh.at[dst_v[b]],
                                            sem_s[b], add=True))
            for b in range(_NBUF):
                s_d[b].wait()
            return 0

        lax.fori_loop(0, base_count // _NBUF, body, 0)

        @pl.when(wid < extra)
        def _():
            base = (base_count * nw + wid) * _CHUNK
            pltpu.sync_copy(src_hbm.at[pl.ds(base, _CHUNK)], src_v[0])
            pltpu.sync_copy(dst_hbm.at[pl.ds(base, _CHUNK)], dst_v[0])
            pltpu.async_copy(x_hbm.at[src_v[0]], rows[0], sem_g[0]).wait()
            pltpu.sync_copy(rows[0], agg_sh.at[dst_v[0]], add=True)

        plsc.subcore_barrier()

        @pl.when(s < ns - 1)
        def _():
            pltpu.sync_copy(
                agg_sh.at[pl.ds(s * _ROWS_PER_TILE, _ROWS_PER_TILE)],
                out_hbm.at[c, pl.ds(s * _ROWS_PER_TILE, _ROWS_PER_TILE)],
            )

        @pl.when(s == ns - 1)
        def _():
            pltpu.sync_copy(
                agg_sh.at[pl.ds((ns - 1) * _ROWS_PER_TILE, tail_rows)],
                out_hbm.at[c, pl.ds((ns - 1) * _ROWS_PER_TILE, tail_rows)],
            )

    return k(x, src, dst)


def _mlp_body(x_ref, p_ref, w1_ref, b1_ref, w2_ref, b2_ref, wg_ref, bg_ref,
              wa_ref, ba_ref, h_ref, a_ref):
    dn = (((1,), (1,)), ((), ()))
    xa = x_ref[...] + p_ref[0] + p_ref[1]
    h1 = lax.dot_general(xa, w1_ref[...], dn, preferred_element_type=jnp.float32)
    h1 = jnp.maximum(h1 + b1_ref[...], 0.0)
    h = lax.dot_general(h1, w2_ref[...], dn, preferred_element_type=jnp.float32)
    h = h + b2_ref[...]
    h_ref[...] = h
    ga = lax.dot_general(h, wg_ref[...], dn, preferred_element_type=jnp.float32)
    ga = jnp.tanh(ga + bg_ref[...])
    alpha = lax.dot_general(wa_ref[...], ga, dn, preferred_element_type=jnp.float32)
    e = jnp.exp(alpha - jnp.max(alpha))
    a_ref[...] = e / jnp.sum(e)


def kernel(x, edge_index, W1, b1, W2, b2, Wg, bg, Wa, ba):
    src = edge_index[0]
    dst = edge_index[1]
    parts = _sc_scatter_add(x, src, dst)
    h, a = pl.pallas_call(
        _mlp_body,
        out_shape=[
            jax.ShapeDtypeStruct((_N, _D), jnp.float32),
            jax.ShapeDtypeStruct((1, _N), jnp.float32),
        ],
    )(x, parts, W1, b1.reshape(1, -1), W2, b2.reshape(1, -1),
      Wg, bg.reshape(1, -1), Wa, ba.reshape(1, 1))
    return h, a[0]


# speedup vs baseline: 2.1085x; 1.0285x over previous
import functools

import jax
import jax.numpy as jnp
from jax import lax
from jax.experimental import pallas as pl
from jax.experimental.pallas import tpu as pltpu
from jax.experimental.pallas import tpu_sc as plsc

_N, _D, _E = 10000, 128, 320000
_CHUNK = 64
_NBUF = 4
_ROWS_PER_TILE = 640


def _sc_scatter_add(x, src, dst):
    info = plsc.get_sparse_core_info()
    nc, ns = info.num_cores, info.num_subcores
    nw = nc * ns
    n_chunks = _E // _CHUNK
    assert _E % _CHUNK == 0
    n_pad = _ROWS_PER_TILE * ns
    zp = _ROWS_PER_TILE // _CHUNK
    assert _ROWS_PER_TILE % _CHUNK == 0
    tail_rows = _N - (ns - 1) * _ROWS_PER_TILE
    assert tail_rows > 0 and (ns - 1) * _ROWS_PER_TILE % 8 == 0

    mesh = plsc.VectorSubcoreMesh(core_axis_name="c", subcore_axis_name="s")

    @functools.partial(
        pl.kernel,
        out_type=jax.ShapeDtypeStruct((nc, _N, _D), jnp.float32),
        mesh=mesh,
        scratch_types=[
            pltpu.MemorySpace.VMEM_SHARED((n_pad, _D), jnp.float32),
            [pltpu.MemorySpace.VMEM((_CHUNK,), jnp.int32)] * _NBUF,
            [pltpu.MemorySpace.VMEM((_CHUNK,), jnp.int32)] * _NBUF,
            [pltpu.MemorySpace.VMEM((_CHUNK, _D), jnp.float32)] * _NBUF,
            [pltpu.SemaphoreType.DMA] * _NBUF,
            [pltpu.SemaphoreType.DMA] * _NBUF,
            [pltpu.SemaphoreType.DMA] * _NBUF,
            [pltpu.SemaphoreType.DMA] * _NBUF,
        ],
    )
    def k(x_hbm, src_hbm, dst_hbm, out_hbm, agg_sh, src_v, dst_v, rows,
          sem_is, sem_id, sem_g, sem_s):
        c = lax.axis_index("c")
        s = lax.axis_index("s")
        wid = s * nc + c

        def zrow(i, _):
            def zlane(j, _):
                rows[0][i, pl.ds(j * 16, 16)] = jnp.zeros((16,), jnp.float32)
                return 0

            lax.fori_loop(0, _D // 16, zlane, 0)
            return 0

        lax.fori_loop(0, _CHUNK, zrow, 0)

        for p in range(zp):
            pltpu.sync_copy(
                rows[0],
                agg_sh.at[pl.ds(s * _ROWS_PER_TILE + p * _CHUNK, _CHUNK)],
            )

        plsc.subcore_barrier()

        base_count = n_chunks // nw
        extra = n_chunks % nw
        assert base_count % _NBUF == 0

        def body(i, _):
            idx_d = []
            for b in range(_NBUF):
                base = ((i * _NBUF + b) * nw + wid) * _CHUNK
                idx_d.append((
                    pltpu.async_copy(src_hbm.at[pl.ds(base, _CHUNK)],
                                     src_v[b], sem_is[b]),
                    pltpu.async_copy(dst_hbm.at[pl.ds(base, _CHUNK)],
                                     dst_v[b], sem_id[b]),
                ))
            g_d = []
            for b in range(_NBUF):
                idx_d[b][0].wait()
                g_d.append(pltpu.async_copy(x_hbm.at[src_v[b]], rows[b],
                                            sem_g[b]))
            s_d = []
            for b in range(_NBUF):
                g_d[b].wait()
                idx_d[b][1].wait()
                s_d.append(pltpu.async_copy(rows[b], agg_sh.at[dst_v[b]],
                                            sem_s[b], add=True))
            for b in range(_NBUF):
                s_d[b].wait()
            return 0

        lax.fori_loop(0, base_count // _NBUF, body, 0)

        @pl.when(wid < extra)
        def _():
            base = (base_count * nw + wid) * _CHUNK
            pltpu.sync_copy(src_hbm.at[pl.ds(base, _CHUNK)], src_v[0])
            pltpu.sync_copy(dst_hbm.at[pl.ds(base, _CHUNK)], dst_v[0])
            pltpu.async_copy(x_hbm.at[src_v[0]], rows[0], sem_g[0]).wait()
            pltpu.sync_copy(rows[0], agg_sh.at[dst_v[0]], add=True)

        plsc.subcore_barrier()

        @pl.when(s < ns - 1)
        def _():
            pltpu.sync_copy(
                agg_sh.at[pl.ds(s * _ROWS_PER_TILE, _ROWS_PER_TILE)],
                out_hbm.at[c, pl.ds(s * _ROWS_PER_TILE, _ROWS_PER_TILE)],
            )

        @pl.when(s == ns - 1)
        def _():
            pltpu.sync_copy(
                agg_sh.at[pl.ds((ns - 1) * _ROWS_PER_TILE, tail_rows)],
                out_hbm.at[c, pl.ds((ns - 1) * _ROWS_PER_TILE, tail_rows)],
            )

    return k(x, src, dst)


def _mlp_body(x_ref, p_ref, w1_ref, b1_ref, w2_ref, b2_ref, wg_ref, bg_ref,
              wa_ref, ba_ref, h_ref, a_ref):
    dn = (((1,), (1,)), ((), ()))
    xa = x_ref[...] + p_ref[0] + p_ref[1]
    h1 = lax.dot_general(xa, w1_ref[...], dn, preferred_element_type=jnp.float32)
    h1 = jnp.maximum(h1 + b1_ref[...], 0.0)
    h = lax.dot_general(h1, w2_ref[...], dn, preferred_element_type=jnp.float32)
    h = h + b2_ref[...]
    h_ref[...] = h
    ga = lax.dot_general(h, wg_ref[...], dn, preferred_element_type=jnp.float32)
    ga = jnp.tanh(ga + bg_ref[...])
    alpha = lax.dot_general(wa_ref[...], ga, dn, preferred_element_type=jnp.float32)
    e = jnp.exp(alpha - jnp.max(alpha))
    a_ref[...] = e / jnp.sum(e)


def kernel(x, edge_index, W1, b1, W2, b2, Wg, bg, Wa, ba):
    src = edge_index[0]
    dst = edge_index[1]
    parts = _sc_scatter_add(x, src, dst)
    h, a = pl.pallas_call(
        _mlp_body,
        out_shape=[
            jax.ShapeDtypeStruct((_N, _D), jnp.float32),
            jax.ShapeDtypeStruct((1, _N), jnp.float32),
        ],
    )(x, parts, W1, b1.reshape(1, -1), W2, b2.reshape(1, -1),
      Wg, bg.reshape(1, -1), Wa, ba.reshape(1, 1))
    return h, a[0]
